# baseline (speedup 1.0000x reference)
import functools

import jax
import jax.numpy as jnp
from jax import lax
from jax.experimental import pallas as pl
from jax.experimental.pallas import tpu as pltpu
from jax.experimental.pallas import tpu_sc as plsc

N = 50000
E = 800000
H = 64
F = 128

NB = 25
BN = N // NB
NSC = 2
NT = 16
CH = 400
EPT = E // NT
NCHUNK = EPT // CH
RPT = 3128
NP = RPT * NT
GW = 16
NBUF = 6


def _leaky(x):
    return jnp.where(x >= 0, x, 0.1 * x)


def _node_embed_body(d_ref, pw1, pb1, pw2, gw1, gb1, gw2, pb2, gb2,
                     h0_ref, hg_ref, gmx_ref):
    i = pl.program_id(0)
    d = d_ref[...]
    h0 = jnp.dot(_leaky(d * pw1[...] + pb1[...]), pw2[...],
                 preferred_element_type=jnp.float32) + pb2[...]
    hg = jnp.dot(_leaky(d * gw1[...] + gb1[...]), gw2[...],
                 preferred_element_type=jnp.float32) + gb2[...]
    h0_ref[...] = h0
    hg_ref[...] = hg
    bmx = jnp.max(h0, axis=0, keepdims=True)

    @pl.when(i == 0)
    def _():
        gmx_ref[...] = bmx

    @pl.when(i > 0)
    def _():
        gmx_ref[...] = jnp.maximum(gmx_ref[...], bmx)


def _node_embed(delay, pi_w1, pi_b1, pi_w2, pi_b2,
                glob_w1, glob_b1, glob_w2, glob_b2):
    full2 = lambda s: pl.BlockSpec(s, lambda i: (0, 0))
    return pl.pallas_call(
        _node_embed_body,
        grid=(NB,),
        in_specs=[
            pl.BlockSpec((BN, 1), lambda i: (i, 0)),
            full2((1, 32)), full2((1, 32)), full2((32, H)),
            full2((1, 32)), full2((1, 32)), full2((32, H)),
            full2((1, H)), full2((1, H)),
        ],
        out_specs=[
            pl.BlockSpec((BN, H), lambda i: (i, 0)),
            pl.BlockSpec((BN, H), lambda i: (i, 0)),
            pl.BlockSpec((1, H), lambda i: (0, 0)),
        ],
        out_shape=[
            jax.ShapeDtypeStruct((N, H), jnp.float32),
            jax.ShapeDtypeStruct((N, H), jnp.float32),
            jax.ShapeDtypeStruct((1, H), jnp.float32),
        ],
    )(delay, pi_w1, pi_b1.reshape(1, 32), pi_w2,
      glob_w1, glob_b1.reshape(1, 32), glob_w2,
      pi_b2.reshape(1, H), glob_b2.reshape(1, H))


RWT = 1564
CHT = 391
NCHT = RWT // CHT


def _sc_tables_body(h0_hbm, gmx_hbm, *refs):
    outs = refs[:8]
    h0v, gmxv = refs[8], refs[9]
    tb = refs[10]
    c = lax.axis_index("c")
    s = lax.axis_index("s")
    w = c * NT + s
    base = jnp.where(w == NT * NSC - 1, N - RWT, w * RWT)
    pltpu.sync_copy(gmx_hbm, gmxv)
    gj = [gmxv[pl.ds(GW * j, GW)] for j in range(4)]

    def chunk(ci, carry):
        r0 = base + ci * CHT
        pltpu.sync_copy(h0_hbm.at[pl.ds(r0, CHT)], h0v)

        def row(r, carry2):
            for j in range(4):
                v = h0v[r, pl.ds(GW * j, GW)]
                e0 = jnp.exp(v - gj[j])
                tb[j][r, :] = e0
                tb[4 + j][r, :] = e0 * v
            return carry2

        lax.fori_loop(0, CHT, row, 0)
        for j in range(8):
            pltpu.sync_copy(tb[j], outs[j].at[pl.ds(r0, CHT)])
        return carry

    lax.fori_loop(0, NCHT, chunk, 0)


def _tables(h0, gmx):
    fn = pl.kernel(
        _sc_tables_body,
        out_type=[jax.ShapeDtypeStruct((N, GW), jnp.float32)] * 8,
        mesh=plsc.VectorSubcoreMesh(**_SC_MESH),
        scratch_types=[
            pltpu.VMEM((CHT, H), jnp.float32),
            pltpu.VMEM((H,), jnp.float32),
            [pltpu.VMEM((CHT, GW), jnp.float32) for _ in range(8)],
        ],
        compiler_params=_SC_PARAMS,
    )
    return fn(h0, gmx.reshape(H))


_SC_PARAMS = pltpu.CompilerParams(use_tc_tiling_on_sc=False,
                                  needs_layout_passes=False)
_SC_MESH = dict(core_axis_name="c", subcore_axis_name="s",
                num_cores=NSC, num_subcores=NT)


def _pipelined_round(nchunk, fill, wait_fill, idx_d, rows, acc, sems):
    def scatter(b):
        pltpu.async_copy(rows[b], acc.at[idx_d[b]], sems[b], add=True)

    def wait_scatter(b):
        pltpu.make_async_copy(rows[b], acc.at[idx_d[b]], sems[b]).wait()

    for b in range(NBUF):
        fill(b, b)

    def step(i, carry):
        k0 = i * NBUF
        for b in range(NBUF):
            k = k0 + b

            @pl.when(k < nchunk)
            def _():
                wait_fill(b)
                scatter(b)

            @pl.when(k < nchunk - NBUF)
            def _():
                wait_scatter(b)
                fill(k + NBUF, b)
        return carry

    lax.fori_loop(0, (nchunk + NBUF - 1) // NBUF, step, 0)
    for b in range(NBUF):
        wait_scatter(b)


def _edge_body(src_hbm, dst_hbm, g0, g1, g2, g3, g4, g5, g6, g7, zg, big_out,
               idx_s, idx_d, rows, acc, semi, semg, sems):
    c = lax.axis_index("c")
    s = lax.axis_index("s")
    row0 = s * RPT
    base0 = s * EPT
    gtbls = [g0, g1, g2, g3, g4, g5, g6, g7]

    def do_round(gi):
        goff = gi * GW
        tbl = gtbls[gi]
        pltpu.sync_copy(zg, acc.at[pl.ds(row0, RPT)])
        plsc.subcore_barrier()

        def fill(k, b):
            off = base0 + k * CH
            pltpu.async_copy(src_hbm.at[pl.ds(off, CH)], idx_s[b], semi[b])
            pltpu.async_copy(dst_hbm.at[pl.ds(off, CH)], idx_d[b], semi[b])
            pltpu.make_async_copy(dst_hbm.at[pl.ds(0, CH)], idx_d[b],
                                  semi[b]).wait()
            pltpu.make_async_copy(src_hbm.at[pl.ds(0, CH)], idx_s[b],
                                  semi[b]).wait()
            pltpu.async_copy(tbl.at[idx_s[b]], rows[b], semg[b])

        def wait_fill(b):
            pltpu.make_async_copy(tbl.at[idx_s[b]], rows[b], semg[b]).wait()

        _pipelined_round(NCHUNK, fill, wait_fill, idx_d, rows, acc, sems)
        plsc.subcore_barrier()
        pltpu.sync_copy(acc.at[pl.ds(row0, RPT)],
                        big_out.at[pl.ds(row0, RPT), pl.ds(goff, GW)])
        plsc.subcore_barrier()

    @pl.when(c == 0)
    def _():
        for r in range(4):
            do_round(r)

    @pl.when(c == 1)
    def _():
        for r in range(4, 8):
            do_round(r)


def _edge_pass(src, dst, gs):
    zg = jnp.zeros((RPT, GW), jnp.float32)
    fn = pl.kernel(
        _edge_body,
        out_type=jax.ShapeDtypeStruct((NP, 2 * H), jnp.float32),
        mesh=plsc.VectorSubcoreMesh(**_SC_MESH),
        scratch_types=[
            [pltpu.VMEM((CH,), jnp.int32) for _ in range(NBUF)],
            [pltpu.VMEM((CH,), jnp.int32) for _ in range(NBUF)],
            [pltpu.VMEM((CH, GW), jnp.float32) for _ in range(NBUF)],
            pltpu.VMEM_SHARED((NP, GW), jnp.float32),
            [pltpu.SemaphoreType.DMA for _ in range(NBUF)],
            [pltpu.SemaphoreType.DMA for _ in range(NBUF)],
            [pltpu.SemaphoreType.DMA for _ in range(NBUF)],
        ],
        compiler_params=_SC_PARAMS,
    )
    return fn(src, dst, *gs, zg)


EPT_B = E // (NSC * NT)
NCHUNK_B = EPT_B // CH


def _bit_body(dst_hbm, bit_hbm, zg, q0_out, q1_out,
              idx_d, bitv, rows, acc, semi, sems):
    c = lax.axis_index("c")
    s = lax.axis_index("s")
    row0 = s * RPT
    base0 = (c * NT + s) * EPT_B

    pltpu.sync_copy(zg, acc.at[pl.ds(row0, RPT)])

    def zrow(i, carry):
        for b in range(NBUF):
            rows[b][i, :] = jnp.zeros((GW,), jnp.float32)
        return carry

    lax.fori_loop(0, CH, zrow, 0)
    plsc.subcore_barrier()

    def fill(k, b):
        off = base0 + k * CH
        pltpu.async_copy(bit_hbm.at[pl.ds(off, CH)], bitv[b], semi[b])
        pltpu.async_copy(dst_hbm.at[pl.ds(off, CH)], idx_d[b], semi[b])
        pltpu.make_async_copy(dst_hbm.at[pl.ds(0, CH)], idx_d[b],
                              semi[b]).wait()
        pltpu.make_async_copy(bit_hbm.at[pl.ds(0, CH)], bitv[b],
                              semi[b]).wait()

        def bexp(i, carry):
            v = bitv[b][pl.ds(i * 16, 16)]
            ev = jnp.exp(v)
            lanes = lax.iota(jnp.int32, 16) + i * 16
            z16 = jnp.zeros((16,), jnp.int32)
            plsc.store_scatter(rows[b], [lanes, z16], ev)
            plsc.store_scatter(rows[b], [lanes, z16 + 1], ev * v)
            return carry

        lax.fori_loop(0, CH // 16, bexp, 0)

    def wait_fill(b):
        pass

    _pipelined_round(NCHUNK_B, fill, wait_fill, idx_d, rows, acc, sems)
    plsc.subcore_barrier()

    @pl.when(c == 0)
    def _():
        pltpu.sync_copy(acc.at[pl.ds(row0, RPT)], q0_out.at[pl.ds(row0, RPT)])

    @pl.when(c == 1)
    def _():
        pltpu.sync_copy(acc.at[pl.ds(row0, RPT)], q1_out.at[pl.ds(row0, RPT)])


def _bit_pass(dst, bits):
    zg = jnp.zeros((RPT, GW), jnp.float32)
    fn = pl.kernel(
        _bit_body,
        out_type=[jax.ShapeDtypeStruct((NP, GW), jnp.float32)] * 2,
        mesh=plsc.VectorSubcoreMesh(**_SC_MESH),
        scratch_types=[
            [pltpu.VMEM((CH,), jnp.int32) for _ in range(NBUF)],
            [pltpu.VMEM((CH,), jnp.float32) for _ in range(NBUF)],
            [pltpu.VMEM((CH, GW), jnp.float32) for _ in range(NBUF)],
            pltpu.VMEM_SHARED((NP, GW), jnp.float32),
            [pltpu.SemaphoreType.DMA for _ in range(NBUF)],
            [pltpu.SemaphoreType.DMA for _ in range(NBUF)],
        ],
        compiler_params=_SC_PARAMS,
    )
    return fn(dst, bits, zg)


def _epilogue_body(big, q0, q1,
                   feat_ref, h0_ref, hg_ref, ipo, imod,
                   gw1, gb1, gw2, gb2, mw1, mb1, mw2, mb2,
                   ow1, ob1, ow2, ob2, out_ref):
    bb = big[...]
    s64 = bb[:, :H]
    t64 = bb[:, H:]
    ng = t64 / (s64 + 1e-9)
    qq = q0[...] + q1[...]
    sb = qq[:, 0:1]
    nb = qq[:, 1:2] / (sb + 1e-9)
    feat = feat_ref[...]

    xg = jnp.concatenate([ng, feat], axis=1)
    hgate = jnp.dot(_leaky(jnp.dot(xg, gw1[...],
                                   preferred_element_type=jnp.float32)
                           + gb1[...]), gw2[...],
                    preferred_element_type=jnp.float32) + gb2[...]
    xm = jnp.concatenate([ng, nb, feat], axis=1)
    hmod = jnp.dot(_leaky(jnp.dot(xm, mw1[...],
                                  preferred_element_type=jnp.float32)
                          + mb1[...]), mw2[...],
                   preferred_element_type=jnp.float32) + mb2[...]
    not_po = ipo[...] != 1
    hgate = jnp.where(not_po, jnp.maximum(hgate, 0.0), hgate)
    hmod = jnp.where(not_po, jnp.maximum(hmod, 0.0), hmod)
    h = jnp.where(imod[...] == 1, hmod, hgate)
    h = jnp.where(sb == 0, h0_ref[...], h)
    xo = jnp.concatenate([h, hg_ref[...]], axis=1)
    out_ref[...] = jnp.dot(_leaky(jnp.dot(xo, ow1[...],
                                          preferred_element_type=jnp.float32)
                                  + ob1[...]), ow2[...],
                           preferred_element_type=jnp.float32) + ob2[...]


def _epilogue(big, q0, q1, feat, h0, hg, is_po, is_module,
              gate_w1, gate_b1, gate_w2, gate_b2,
              mod_w1, mod_b1, mod_w2, mod_b2,
              out_w1, out_b1, out_w2, out_b2):
    full2 = lambda a: pl.BlockSpec(a.shape, lambda i: (0, 0))
    ws = [gate_w1, gate_b1.reshape(1, -1), gate_w2, gate_b2.reshape(1, -1),
          mod_w1, mod_b1.reshape(1, -1), mod_w2, mod_b2.reshape(1, -1),
          out_w1, out_b1.reshape(1, -1), out_w2, out_b2.reshape(1, -1)]
    return pl.pallas_call(
        _epilogue_body,
        grid=(NB,),
        in_specs=[pl.BlockSpec((BN, 2 * H), lambda i: (i, 0)),
                  pl.BlockSpec((BN, GW), lambda i: (i, 0)),
                  pl.BlockSpec((BN, GW), lambda i: (i, 0)),
                  pl.BlockSpec((BN, F), lambda i: (i, 0)),
                  pl.BlockSpec((BN, H), lambda i: (i, 0)),
                  pl.BlockSpec((BN, H), lambda i: (i, 0)),
                  pl.BlockSpec((BN, 1), lambda i: (i, 0)),
                  pl.BlockSpec((BN, 1), lambda i: (i, 0))]
        + [full2(a) for a in ws],
        out_specs=pl.BlockSpec((BN, 1), lambda i: (i, 0)),
        out_shape=jax.ShapeDtypeStruct((N, 1), jnp.float32),
    )(big, q0, q1, feat, h0, hg,
      is_po.reshape(N, 1), is_module.reshape(N, 1), *ws)


def kernel(feat, delay, bit_position, pi_w1, pi_b1, pi_w2, pi_b2,
           gate_w1, gate_b1, gate_w2, gate_b2,
           mod_w1, mod_b1, mod_w2, mod_b2,
           glob_w1, glob_b1, glob_w2, glob_b2,
           out_w1, out_b1, out_w2, out_b2,
           edge_index, is_po, is_module):
    src = edge_index[0]
    dst = edge_index[1]

    q0, q1 = _bit_pass(dst, bit_position)
    h0, hg, gmx = _node_embed(delay, pi_w1, pi_b1, pi_w2, pi_b2,
                              glob_w1, glob_b1, glob_w2, glob_b2)
    gs = _tables(h0, gmx)
    big = _edge_pass(src, dst, gs)

    return _epilogue(big, q0, q1, feat, h0, hg, is_po, is_module,
                     gate_w1, gate_b1, gate_w2, gate_b2,
                     mod_w1, mod_b1, mod_w2, mod_b2,
                     out_w1, out_b1, out_w2, out_b2)


# speedup vs baseline: 1.2442x; 1.2442x over previous
import functools

import jax
import jax.numpy as jnp
from jax import lax
from jax.experimental import pallas as pl
from jax.experimental.pallas import tpu as pltpu
from jax.experimental.pallas import tpu_sc as plsc

N = 50000
E = 800000
H = 64
F = 128

NB = 25
BN = N // NB
NSC = 2
NT = 16
CH = 1000
EPT = E // NT
NCHUNK = EPT // CH
RPT = 3128
NP = RPT * NT
GW = 16
NBUF = 4


def _leaky(x):
    return jnp.where(x >= 0, x, 0.1 * x)


def _node_embed_body(d_ref, pw1, pb1, pw2, gw1, gb1, gw2, pb2, gb2,
                     h0_ref, hg_ref, gmx_ref):
    i = pl.program_id(0)
    d = d_ref[...]
    h0 = jnp.dot(_leaky(d * pw1[...] + pb1[...]), pw2[...],
                 preferred_element_type=jnp.float32) + pb2[...]
    hg = jnp.dot(_leaky(d * gw1[...] + gb1[...]), gw2[...],
                 preferred_element_type=jnp.float32) + gb2[...]
    h0_ref[...] = h0
    hg_ref[...] = hg
    bmx = jnp.max(h0, axis=0, keepdims=True)

    @pl.when(i == 0)
    def _():
        gmx_ref[...] = bmx

    @pl.when(i > 0)
    def _():
        gmx_ref[...] = jnp.maximum(gmx_ref[...], bmx)


def _node_embed(delay, pi_w1, pi_b1, pi_w2, pi_b2,
                glob_w1, glob_b1, glob_w2, glob_b2):
    full2 = lambda s: pl.BlockSpec(s, lambda i: (0, 0))
    return pl.pallas_call(
        _node_embed_body,
        grid=(NB,),
        in_specs=[
            pl.BlockSpec((BN, 1), lambda i: (i, 0)),
            full2((1, 32)), full2((1, 32)), full2((32, H)),
            full2((1, 32)), full2((1, 32)), full2((32, H)),
            full2((1, H)), full2((1, H)),
        ],
        out_specs=[
            pl.BlockSpec((BN, H), lambda i: (i, 0)),
            pl.BlockSpec((BN, H), lambda i: (i, 0)),
            pl.BlockSpec((1, H), lambda i: (0, 0)),
        ],
        out_shape=[
            jax.ShapeDtypeStruct((N, H), jnp.float32),
            jax.ShapeDtypeStruct((N, H), jnp.float32),
            jax.ShapeDtypeStruct((1, H), jnp.float32),
        ],
    )(delay, pi_w1, pi_b1.reshape(1, 32), pi_w2,
      glob_w1, glob_b1.reshape(1, 32), glob_w2,
      pi_b2.reshape(1, H), glob_b2.reshape(1, H))


RWT = 1564
CHT = 391
NCHT = RWT // CHT


def _sc_tables_body(h0_hbm, gmx_hbm, *refs):
    outs = refs[:8]
    h0v, gmxv = refs[8], refs[9]
    tb = refs[10]
    c = lax.axis_index("c")
    s = lax.axis_index("s")
    w = c * NT + s
    base = jnp.where(w == NT * NSC - 1, N - RWT, w * RWT)
    pltpu.sync_copy(gmx_hbm, gmxv)
    gj = [gmxv[pl.ds(GW * j, GW)] for j in range(4)]

    def chunk(ci, carry):
        r0 = base + ci * CHT
        pltpu.sync_copy(h0_hbm.at[pl.ds(r0, CHT)], h0v)

        def row(r, carry2):
            for j in range(4):
                v = h0v[r, pl.ds(GW * j, GW)]
                e0 = jnp.exp(v - gj[j])
                tb[j][r, :] = e0
                tb[4 + j][r, :] = e0 * v
            return carry2

        lax.fori_loop(0, CHT, row, 0)
        for j in range(8):
            pltpu.sync_copy(tb[j], outs[j].at[pl.ds(r0, CHT)])
        return carry

    lax.fori_loop(0, NCHT, chunk, 0)


def _tables(h0, gmx):
    fn = pl.kernel(
        _sc_tables_body,
        out_type=[jax.ShapeDtypeStruct((N, GW), jnp.float32)] * 8,
        mesh=plsc.VectorSubcoreMesh(**_SC_MESH),
        scratch_types=[
            pltpu.VMEM((CHT, H), jnp.float32),
            pltpu.VMEM((H,), jnp.float32),
            [pltpu.VMEM((CHT, GW), jnp.float32) for _ in range(8)],
        ],
        compiler_params=_SC_PARAMS,
    )
    return fn(h0, gmx.reshape(H))


_SC_PARAMS = pltpu.CompilerParams(use_tc_tiling_on_sc=False,
                                  needs_layout_passes=False)
_SC_MESH = dict(core_axis_name="c", subcore_axis_name="s",
                num_cores=NSC, num_subcores=NT)


def _pipelined_round(nchunk, fill, wait_fill, idx_d, rows, acc, sems):
    def scatter(b):
        pltpu.async_copy(rows[b], acc.at[idx_d[b]], sems[b], add=True)

    def wait_scatter(b):
        pltpu.make_async_copy(rows[b], acc.at[idx_d[b]], sems[b]).wait()

    for b in range(NBUF):
        fill(b, b)

    def step(i, carry):
        k0 = i * NBUF
        for b in range(NBUF):
            k = k0 + b

            @pl.when(k < nchunk)
            def _():
                wait_fill(b)
                scatter(b)

            @pl.when(k < nchunk - NBUF)
            def _():
                wait_scatter(b)
                fill(k + NBUF, b)
        return carry

    lax.fori_loop(0, (nchunk + NBUF - 1) // NBUF, step, 0)
    for b in range(NBUF):
        wait_scatter(b)


def _edge_body(src_hbm, dst_hbm, g0, g1, g2, g3, g4, g5, g6, g7, zg, big_out,
               idx_s, idx_d, rows, acc, semi, semg, sems):
    c = lax.axis_index("c")
    s = lax.axis_index("s")
    row0 = s * RPT
    base0 = s * EPT
    gtbls = [g0, g1, g2, g3, g4, g5, g6, g7]

    def do_round(gi):
        goff = gi * GW
        tbl = gtbls[gi]
        pltpu.sync_copy(zg, acc.at[pl.ds(row0, RPT)])
        plsc.subcore_barrier()

        def fill(k, b):
            off = base0 + k * CH
            pltpu.async_copy(src_hbm.at[pl.ds(off, CH)], idx_s[b], semi[b])
            pltpu.async_copy(dst_hbm.at[pl.ds(off, CH)], idx_d[b], semi[b])
            pltpu.make_async_copy(dst_hbm.at[pl.ds(0, CH)], idx_d[b],
                                  semi[b]).wait()
            pltpu.make_async_copy(src_hbm.at[pl.ds(0, CH)], idx_s[b],
                                  semi[b]).wait()
            pltpu.async_copy(tbl.at[idx_s[b]], rows[b], semg[b])

        def wait_fill(b):
            pltpu.make_async_copy(tbl.at[idx_s[b]], rows[b], semg[b]).wait()

        _pipelined_round(NCHUNK, fill, wait_fill, idx_d, rows, acc, sems)
        plsc.subcore_barrier()
        pltpu.sync_copy(acc.at[pl.ds(row0, RPT)],
                        big_out.at[pl.ds(row0, RPT), pl.ds(goff, GW)])
        plsc.subcore_barrier()

    @pl.when(c == 0)
    def _():
        for r in range(4):
            do_round(r)

    @pl.when(c == 1)
    def _():
        for r in range(4, 8):
            do_round(r)


def _edge_pass(src, dst, gs):
    zg = jnp.zeros((RPT, GW), jnp.float32)
    fn = pl.kernel(
        _edge_body,
        out_type=jax.ShapeDtypeStruct((NP, 2 * H), jnp.float32),
        mesh=plsc.VectorSubcoreMesh(**_SC_MESH),
        scratch_types=[
            [pltpu.VMEM((CH,), jnp.int32) for _ in range(NBUF)],
            [pltpu.VMEM((CH,), jnp.int32) for _ in range(NBUF)],
            [pltpu.VMEM((CH, GW), jnp.float32) for _ in range(NBUF)],
            pltpu.VMEM_SHARED((NP, GW), jnp.float32),
            [pltpu.SemaphoreType.DMA for _ in range(NBUF)],
            [pltpu.SemaphoreType.DMA for _ in range(NBUF)],
            [pltpu.SemaphoreType.DMA for _ in range(NBUF)],
        ],
        compiler_params=_SC_PARAMS,
    )
    return fn(src, dst, *gs, zg)


EPT_B = E // (NSC * NT)
NCHUNK_B = EPT_B // CH


def _bit_body(dst_hbm, bit_hbm, zg, q0_out, q1_out,
              idx_d, bitv, rows, acc, semi, sems):
    c = lax.axis_index("c")
    s = lax.axis_index("s")
    row0 = s * RPT
    base0 = (c * NT + s) * EPT_B

    pltpu.sync_copy(zg, acc.at[pl.ds(row0, RPT)])

    def zrow(i, carry):
        for b in range(NBUF):
            rows[b][i, :] = jnp.zeros((GW,), jnp.float32)
        return carry

    lax.fori_loop(0, CH, zrow, 0)
    plsc.subcore_barrier()

    def fill(k, b):
        off = base0 + k * CH
        pltpu.async_copy(bit_hbm.at[pl.ds(off, CH)], bitv[b], semi[b])
        pltpu.async_copy(dst_hbm.at[pl.ds(off, CH)], idx_d[b], semi[b])
        pltpu.make_async_copy(dst_hbm.at[pl.ds(0, CH)], idx_d[b],
                              semi[b]).wait()
        pltpu.make_async_copy(bit_hbm.at[pl.ds(0, CH)], bitv[b],
                              semi[b]).wait()

        def bexp(i, carry):
            v = bitv[b][pl.ds(i * 16, 16)]
            ev = jnp.exp(v)
            lanes = lax.iota(jnp.int32, 16) + i * 16
            z16 = jnp.zeros((16,), jnp.int32)
            plsc.store_scatter(rows[b], [lanes, z16], ev)
            plsc.store_scatter(rows[b], [lanes, z16 + 1], ev * v)
            return carry

        lax.fori_loop(0, CH // 16, bexp, 0)

    def wait_fill(b):
        pass

    _pipelined_round(NCHUNK_B, fill, wait_fill, idx_d, rows, acc, sems)
    plsc.subcore_barrier()

    @pl.when(c == 0)
    def _():
        pltpu.sync_copy(acc.at[pl.ds(row0, RPT)], q0_out.at[pl.ds(row0, RPT)])

    @pl.when(c == 1)
    def _():
        pltpu.sync_copy(acc.at[pl.ds(row0, RPT)], q1_out.at[pl.ds(row0, RPT)])


def _bit_pass(dst, bits):
    zg = jnp.zeros((RPT, GW), jnp.float32)
    fn = pl.kernel(
        _bit_body,
        out_type=[jax.ShapeDtypeStruct((NP, GW), jnp.float32)] * 2,
        mesh=plsc.VectorSubcoreMesh(**_SC_MESH),
        scratch_types=[
            [pltpu.VMEM((CH,), jnp.int32) for _ in range(NBUF)],
            [pltpu.VMEM((CH,), jnp.float32) for _ in range(NBUF)],
            [pltpu.VMEM((CH, GW), jnp.float32) for _ in range(NBUF)],
            pltpu.VMEM_SHARED((NP, GW), jnp.float32),
            [pltpu.SemaphoreType.DMA for _ in range(NBUF)],
            [pltpu.SemaphoreType.DMA for _ in range(NBUF)],
        ],
        compiler_params=_SC_PARAMS,
    )
    return fn(dst, bits, zg)


def _epilogue_body(big, q0, q1,
                   feat_ref, h0_ref, hg_ref, ipo, imod,
                   gw1, gb1, gw2, gb2, mw1, mb1, mw2, mb2,
                   ow1, ob1, ow2, ob2, out_ref):
    bb = big[...]
    s64 = bb[:, :H]
    t64 = bb[:, H:]
    ng = t64 / (s64 + 1e-9)
    qq = q0[...] + q1[...]
    sb = qq[:, 0:1]
    nb = qq[:, 1:2] / (sb + 1e-9)
    feat = feat_ref[...]

    xg = jnp.concatenate([ng, feat], axis=1)
    hgate = jnp.dot(_leaky(jnp.dot(xg, gw1[...],
                                   preferred_element_type=jnp.float32)
                           + gb1[...]), gw2[...],
                    preferred_element_type=jnp.float32) + gb2[...]
    xm = jnp.concatenate([ng, nb, feat], axis=1)
    hmod = jnp.dot(_leaky(jnp.dot(xm, mw1[...],
                                  preferred_element_type=jnp.float32)
                          + mb1[...]), mw2[...],
                   preferred_element_type=jnp.float32) + mb2[...]
    not_po = ipo[...] != 1
    hgate = jnp.where(not_po, jnp.maximum(hgate, 0.0), hgate)
    hmod = jnp.where(not_po, jnp.maximum(hmod, 0.0), hmod)
    h = jnp.where(imod[...] == 1, hmod, hgate)
    h = jnp.where(sb == 0, h0_ref[...], h)
    xo = jnp.concatenate([h, hg_ref[...]], axis=1)
    out_ref[...] = jnp.dot(_leaky(jnp.dot(xo, ow1[...],
                                          preferred_element_type=jnp.float32)
                                  + ob1[...]), ow2[...],
                           preferred_element_type=jnp.float32) + ob2[...]


def _epilogue(big, q0, q1, feat, h0, hg, is_po, is_module,
              gate_w1, gate_b1, gate_w2, gate_b2,
              mod_w1, mod_b1, mod_w2, mod_b2,
              out_w1, out_b1, out_w2, out_b2):
    full2 = lambda a: pl.BlockSpec(a.shape, lambda i: (0, 0))
    ws = [gate_w1, gate_b1.reshape(1, -1), gate_w2, gate_b2.reshape(1, -1),
          mod_w1, mod_b1.reshape(1, -1), mod_w2, mod_b2.reshape(1, -1),
          out_w1, out_b1.reshape(1, -1), out_w2, out_b2.reshape(1, -1)]
    return pl.pallas_call(
        _epilogue_body,
        grid=(NB,),
        in_specs=[pl.BlockSpec((BN, 2 * H), lambda i: (i, 0)),
                  pl.BlockSpec((BN, GW), lambda i: (i, 0)),
                  pl.BlockSpec((BN, GW), lambda i: (i, 0)),
                  pl.BlockSpec((BN, F), lambda i: (i, 0)),
                  pl.BlockSpec((BN, H), lambda i: (i, 0)),
                  pl.BlockSpec((BN, H), lambda i: (i, 0)),
                  pl.BlockSpec((BN, 1), lambda i: (i, 0)),
                  pl.BlockSpec((BN, 1), lambda i: (i, 0))]
        + [full2(a) for a in ws],
        out_specs=pl.BlockSpec((BN, 1), lambda i: (i, 0)),
        out_shape=jax.ShapeDtypeStruct((N, 1), jnp.float32),
    )(big, q0, q1, feat, h0, hg,
      is_po.reshape(N, 1), is_module.reshape(N, 1), *ws)


def kernel(feat, delay, bit_position, pi_w1, pi_b1, pi_w2, pi_b2,
           gate_w1, gate_b1, gate_w2, gate_b2,
           mod_w1, mod_b1, mod_w2, mod_b2,
           glob_w1, glob_b1, glob_w2, glob_b2,
           out_w1, out_b1, out_w2, out_b2,
           edge_index, is_po, is_module):
    src = edge_index[0]
    dst = edge_index[1]

    q0, q1 = _bit_pass(dst, bit_position)
    h0, hg, gmx = _node_embed(delay, pi_w1, pi_b1, pi_w2, pi_b2,
                              glob_w1, glob_b1, glob_w2, glob_b2)
    gs = _tables(h0, gmx)
    big = _edge_pass(src, dst, gs)

    return _epilogue(big, q0, q1, feat, h0, hg, is_po, is_module,
                     gate_w1, gate_b1, gate_w2, gate_b2,
                     mod_w1, mod_b1, mod_w2, mod_b2,
                     out_w1, out_b1, out_w2, out_b2)
